# baseline (speedup 1.0000x reference)
import functools

import jax
import jax.numpy as jnp
from jax.experimental import pallas as pl
from jax.experimental.pallas import tpu as pltpu

_INTERPRET = False

B = 2
HW = 32
L = HW * HW
C = 384
DI = 768
N = 16
R = 24
EPS = 1e-5


def _ln(x, w, b):
    m = jnp.mean(x, axis=-1, keepdims=True)
    v = jnp.mean((x - m) ** 2, axis=-1, keepdims=True)
    return (x - m) * jax.lax.rsqrt(v + EPS) * w + b


def _silu(x):
    return x / (1.0 + jnp.exp(-x))


def _softplus(x):
    return jnp.maximum(x, 0.0) + jnp.log1p(jnp.exp(-jnp.abs(x)))


def _dotT(a, b):
    return jax.lax.dot_general(a, b, (((1,), (1,)), ((), ())),
                               preferred_element_type=jnp.float32)


def _pe_kernel(xf_ref, w_ref, nw_ref, nb_ref, o_ref):
    t = _dotT(xf_ref[...], w_ref[...])
    nw = nw_ref[...]
    nb = nb_ref[...]
    for g in range(4):
        seg = t[:, g * C:(g + 1) * C]
        o_ref[:, g * C:(g + 1) * C] = _ln(seg, nw, nb)


def _patch_expand(xf, pe_w, pe_nw, pe_nb):
    return pl.pallas_call(
        _pe_kernel,
        out_shape=jax.ShapeDtypeStruct((B * 256, 2 * C * 2), jnp.float32),
        name="pe_expand_ln",
        interpret=_INTERPRET,
    )(xf, pe_w, pe_nw.reshape(1, C), pe_nb.reshape(1, C))


def _proj_kernel(a_ref, s_ref, w1_ref, w2_ref, b_ref, o_ref):
    o_ref[...] = (_dotT(a_ref[...], w1_ref[...]) +
                  _dotT(s_ref[...], w2_ref[...]) + b_ref[...])


def _concat_proj(xh_ps, skip_f, lp_w, lp_b):
    w1 = lp_w[:, :C]
    w2 = lp_w[:, C:]
    return pl.pallas_call(
        _proj_kernel,
        out_shape=jax.ShapeDtypeStruct((B * L, C), jnp.float32),
        name="concat_proj",
        interpret=_INTERPRET,
    )(xh_ps, skip_f, w1, w2, lp_b.reshape(1, C))


def _inproj_kernel(xh_ref, lnw_ref, lnb_ref, w_ref, xc_ref, z_ref):
    h = _ln(xh_ref[...], lnw_ref[...], lnb_ref[...])
    xz = _dotT(h, w_ref[...])
    xc_ref[...] = xz[:, :DI]
    z_ref[...] = xz[:, DI:]


def _ln_inproj(xh, ln_w, ln_b, in_w):
    n_tiles = 8
    rows = B * L // n_tiles
    return pl.pallas_call(
        _inproj_kernel,
        grid=(n_tiles,),
        in_specs=[
            pl.BlockSpec((rows, C), lambda i: (i, 0)),
            pl.BlockSpec((1, C), lambda i: (0, 0)),
            pl.BlockSpec((1, C), lambda i: (0, 0)),
            pl.BlockSpec((2 * DI, C), lambda i: (0, 0)),
        ],
        out_specs=[
            pl.BlockSpec((rows, DI), lambda i: (i, 0)),
            pl.BlockSpec((rows, DI), lambda i: (i, 0)),
        ],
        out_shape=[
            jax.ShapeDtypeStruct((B * L, DI), jnp.float32),
            jax.ShapeDtypeStruct((B * L, DI), jnp.float32),
        ],
        name="ln_inproj",
        interpret=_INTERPRET,
    )(xh, ln_w.reshape(1, C), ln_b.reshape(1, C), in_w)


def _conv_kernel(x_ref, w_ref, b_ref, o_ref):
    x = x_ref[0]
    col = jax.lax.broadcasted_iota(jnp.int32, (L, 1), 0) % HW
    acc = b_ref[...] + jnp.zeros((L, DI), jnp.float32)
    for di in (-1, 0, 1):
        for dj in (-1, 0, 1):
            delta = HW * di + dj
            if delta > 0:
                sh = jnp.concatenate(
                    [x[delta:], jnp.zeros((delta, DI), jnp.float32)], axis=0)
            elif delta < 0:
                sh = jnp.concatenate(
                    [jnp.zeros((-delta, DI), jnp.float32), x[:delta]], axis=0)
            else:
                sh = x
            if dj == 1:
                sh = jnp.where(col != HW - 1, sh, 0.0)
            elif dj == -1:
                sh = jnp.where(col != 0, sh, 0.0)
            acc = acc + w_ref[(di + 1) * 3 + (dj + 1)][None, :] * sh
    o_ref[0] = _silu(acc)


def _conv_silu(xc, conv_w9, conv_b):
    return pl.pallas_call(
        _conv_kernel,
        grid=(B,),
        in_specs=[
            pl.BlockSpec((1, L, DI), lambda b: (b, 0, 0)),
            pl.BlockSpec((9, DI), lambda b: (0, 0)),
            pl.BlockSpec((1, DI), lambda b: (0, 0)),
        ],
        out_specs=pl.BlockSpec((1, L, DI), lambda b: (b, 0, 0)),
        out_shape=jax.ShapeDtypeStruct((B, L, DI), jnp.float32),
        name="dwconv_silu",
        interpret=_INTERPRET,
    )(xc.reshape(B, L, DI), conv_w9, conv_b.reshape(1, DI))


T = 256
LOG2E = 1.4426950408889634


def _scan_loop(o_ref, dt_s, du_s, bc_s, a2_s, *, backward):
    order = tuple(reversed(range(T))) if backward else tuple(range(T))

    def body(c, h):
        cc = (L // T - 1 - c) if backward else c
        base = pl.multiple_of(cc * T, T)
        A2 = a2_s[...]
        dtc = dt_s[pl.ds(base, T), :]
        duc = du_s[pl.ds(base, T), :]
        bcT = bc_s[pl.ds(base, T), :].T
        rows = [None] * T
        for s in order:
            a = jnp.exp2(A2 * dtc[s:s + 1])
            x = bcT[:N, s:s + 1] * duc[s:s + 1]
            h = a * h + x
            rows[s] = jnp.sum(h * bcT[N:, s:s + 1], axis=0, keepdims=True)
        o_ref[0, 0, pl.ds(base, T), :] = jnp.concatenate(rows, axis=0)
        return h

    jax.lax.fori_loop(0, L // T, body, jnp.zeros((N, DI), jnp.float32))


def _scan_kernel(seq_ref, xw_ref, dtw_ref, dtb_ref, alog_ref, o_ref,
                 dt_s, du_s, bc_s, a2_s):
    seq = seq_ref[0, 0]
    xdbl = _dotT(seq, xw_ref[0])
    dtr = xdbl[:, :R]
    bc_s[...] = xdbl[:, R:]
    dt = _softplus(_dotT(dtr, dtw_ref[0]) + dtb_ref[0])
    dt_s[...] = dt
    du_s[...] = dt * seq
    a2_s[...] = -LOG2E * jnp.exp(alog_ref[0])

    j = pl.program_id(1)

    @pl.when(j % 2 == 0)
    def _():
        _scan_loop(o_ref, dt_s, du_s, bc_s, a2_s, backward=False)

    @pl.when(j % 2 == 1)
    def _():
        _scan_loop(o_ref, dt_s, du_s, bc_s, a2_s, backward=True)


def _scan(seqs, x_proj_w, dt_proj_w, dt_proj_b, a_log_t):
    wmap = lambda b, j: ((j % 2) * 2 + j // 2, 0, 0)
    return pl.pallas_call(
        _scan_kernel,
        grid=(B, 4),
        in_specs=[
            pl.BlockSpec((1, 1, L, DI), lambda b, j: (b, j // 2, 0, 0)),
            pl.BlockSpec((1, R + 2 * N, DI), wmap),
            pl.BlockSpec((1, DI, R), wmap),
            pl.BlockSpec((1, 1, DI), wmap),
            pl.BlockSpec((1, N, DI), wmap),
        ],
        out_specs=pl.BlockSpec((1, 1, L, DI), lambda b, j: (b, j, 0, 0)),
        out_shape=jax.ShapeDtypeStruct((B, 4, L, DI), jnp.float32),
        scratch_shapes=[
            pltpu.VMEM((L, DI), jnp.float32),
            pltpu.VMEM((L, DI), jnp.float32),
            pltpu.VMEM((L, 2 * N), jnp.float32),
            pltpu.VMEM((N, DI), jnp.float32),
        ],
        compiler_params=pltpu.CompilerParams(
            dimension_semantics=("arbitrary", "arbitrary"),
            vmem_limit_bytes=50 * 1024 * 1024,
        ),
        name="ss2d_scan",
        interpret=_INTERPRET,
    )(seqs, x_proj_w, dt_proj_w, dt_proj_b, a_log_t)


def _combine_kernel(yr_ref, yt_ref, xc_ref, ds_ref, z_ref, xh_ref, onw_ref,
                    onb_ref, ow_ref, o_ref):
    ds_sum = jnp.sum(ds_ref[...], axis=0, keepdims=True)
    y = yr_ref[...] + yt_ref[...] + ds_sum * xc_ref[...]
    y = _ln(y, onw_ref[...], onb_ref[...])
    y = y * _silu(z_ref[...])
    o_ref[...] = xh_ref[...] + _dotT(y, ow_ref[...])


def _combine(y_r, y_t, xc, ds, z, xh, onw, onb, ow):
    return pl.pallas_call(
        _combine_kernel,
        grid=(B,),
        in_specs=[
            pl.BlockSpec((L, DI), lambda b: (b, 0)),
            pl.BlockSpec((L, DI), lambda b: (b, 0)),
            pl.BlockSpec((L, DI), lambda b: (b, 0)),
            pl.BlockSpec((4, DI), lambda b: (0, 0)),
            pl.BlockSpec((L, DI), lambda b: (b, 0)),
            pl.BlockSpec((L, C), lambda b: (b, 0)),
            pl.BlockSpec((1, DI), lambda b: (0, 0)),
            pl.BlockSpec((1, DI), lambda b: (0, 0)),
            pl.BlockSpec((C, DI), lambda b: (0, 0)),
        ],
        out_specs=pl.BlockSpec((L, C), lambda b: (b, 0)),
        out_shape=jax.ShapeDtypeStruct((B * L, C), jnp.float32),
        name="combine_out",
        interpret=_INTERPRET,
    )(y_r, y_t, xc, ds, z, xh, onw.reshape(1, DI), onb.reshape(1, DI), ow)


def _spatial_t(a):
    return a.reshape(B, HW, HW, DI).transpose(0, 2, 1, 3).reshape(B, L, DI)


def kernel(x, skip, pe_expand_w, pe_norm_w, pe_norm_b, lp_w, lp_b,
           blk_ln_w, blk_ln_b, in_proj_w, conv_w, conv_b, x_proj_w,
           dt_proj_w, dt_proj_b, A_log, Ds, out_norm_w, out_norm_b,
           out_proj_w):
    H = x.shape[2]
    xf = x.transpose(0, 2, 3, 1).reshape(B * H * H, 2 * C)
    pe = _patch_expand(xf, pe_expand_w, pe_norm_w, pe_norm_b)
    xh_ps = (pe.reshape(B, H, H, 2, 2, C).transpose(0, 1, 3, 2, 4, 5)
             .reshape(B * L, C))
    skip_f = skip.transpose(0, 2, 3, 1).reshape(B * L, C)
    xh = _concat_proj(xh_ps, skip_f, lp_w, lp_b)

    depth = blk_ln_w.shape[0]
    for i in range(depth):
        xc, z = _ln_inproj(xh, blk_ln_w[i], blk_ln_b[i], in_proj_w[i])
        xc = _conv_silu(xc, conv_w[i].reshape(DI, 9).T, conv_b[i])
        seqs = jnp.stack([xc, _spatial_t(xc)], axis=1)
        ys = _scan(seqs, x_proj_w[i], dt_proj_w[i],
                   dt_proj_b[i].reshape(4, 1, DI),
                   A_log[i].transpose(0, 2, 1))
        y_r = (ys[:, 0] + ys[:, 1]).reshape(B * L, DI)
        y_t = _spatial_t(ys[:, 2] + ys[:, 3]).reshape(B * L, DI)
        xh = _combine(y_r, y_t, xc.reshape(B * L, DI), Ds[i], z, xh,
                      out_norm_w[i], out_norm_b[i], out_proj_w[i])

    return xh.reshape(B, HW, HW, C).transpose(0, 3, 1, 2)


# speedup vs baseline: 1.0284x; 1.0284x over previous
import functools

import jax
import jax.numpy as jnp
from jax.experimental import pallas as pl
from jax.experimental.pallas import tpu as pltpu

_INTERPRET = False

B = 2
HW = 32
L = HW * HW
C = 384
DI = 768
N = 16
R = 24
EPS = 1e-5


def _ln(x, w, b):
    m = jnp.mean(x, axis=-1, keepdims=True)
    v = jnp.mean((x - m) ** 2, axis=-1, keepdims=True)
    return (x - m) * jax.lax.rsqrt(v + EPS) * w + b


def _silu(x):
    return x / (1.0 + jnp.exp(-x))


def _softplus(x):
    return jnp.maximum(x, 0.0) + jnp.log1p(jnp.exp(-jnp.abs(x)))


def _dotT(a, b):
    return jax.lax.dot_general(a, b, (((1,), (1,)), ((), ())),
                               preferred_element_type=jnp.float32)


def _pe_kernel(xf_ref, w_ref, nw_ref, nb_ref, o_ref):
    t = _dotT(xf_ref[...], w_ref[...])
    nw = nw_ref[...]
    nb = nb_ref[...]
    for g in range(4):
        seg = t[:, g * C:(g + 1) * C]
        o_ref[:, g * C:(g + 1) * C] = _ln(seg, nw, nb)


def _patch_expand(xf, pe_w, pe_nw, pe_nb):
    return pl.pallas_call(
        _pe_kernel,
        out_shape=jax.ShapeDtypeStruct((B * 256, 2 * C * 2), jnp.float32),
        name="pe_expand_ln",
        interpret=_INTERPRET,
    )(xf, pe_w, pe_nw.reshape(1, C), pe_nb.reshape(1, C))


def _proj_kernel(a_ref, s_ref, w1_ref, w2_ref, b_ref, o_ref):
    o_ref[...] = (_dotT(a_ref[...], w1_ref[...]) +
                  _dotT(s_ref[...], w2_ref[...]) + b_ref[...])


def _concat_proj(xh_ps, skip_f, lp_w, lp_b):
    w1 = lp_w[:, :C]
    w2 = lp_w[:, C:]
    return pl.pallas_call(
        _proj_kernel,
        out_shape=jax.ShapeDtypeStruct((B * L, C), jnp.float32),
        name="concat_proj",
        interpret=_INTERPRET,
    )(xh_ps, skip_f, w1, w2, lp_b.reshape(1, C))


def _inproj_kernel(xh_ref, lnw_ref, lnb_ref, w_ref, xc_ref, z_ref):
    h = _ln(xh_ref[...], lnw_ref[...], lnb_ref[...])
    xz = _dotT(h, w_ref[...])
    xc_ref[...] = xz[:, :DI]
    z_ref[...] = xz[:, DI:]


def _ln_inproj(xh, ln_w, ln_b, in_w):
    n_tiles = 8
    rows = B * L // n_tiles
    return pl.pallas_call(
        _inproj_kernel,
        grid=(n_tiles,),
        in_specs=[
            pl.BlockSpec((rows, C), lambda i: (i, 0)),
            pl.BlockSpec((1, C), lambda i: (0, 0)),
            pl.BlockSpec((1, C), lambda i: (0, 0)),
            pl.BlockSpec((2 * DI, C), lambda i: (0, 0)),
        ],
        out_specs=[
            pl.BlockSpec((rows, DI), lambda i: (i, 0)),
            pl.BlockSpec((rows, DI), lambda i: (i, 0)),
        ],
        out_shape=[
            jax.ShapeDtypeStruct((B * L, DI), jnp.float32),
            jax.ShapeDtypeStruct((B * L, DI), jnp.float32),
        ],
        name="ln_inproj",
        interpret=_INTERPRET,
    )(xh, ln_w.reshape(1, C), ln_b.reshape(1, C), in_w)


def _conv_kernel(x_ref, w_ref, b_ref, o_ref):
    x = x_ref[0]
    col = jax.lax.broadcasted_iota(jnp.int32, (L, 1), 0) % HW
    acc = b_ref[...] + jnp.zeros((L, DI), jnp.float32)
    for di in (-1, 0, 1):
        for dj in (-1, 0, 1):
            delta = HW * di + dj
            if delta > 0:
                sh = jnp.concatenate(
                    [x[delta:], jnp.zeros((delta, DI), jnp.float32)], axis=0)
            elif delta < 0:
                sh = jnp.concatenate(
                    [jnp.zeros((-delta, DI), jnp.float32), x[:delta]], axis=0)
            else:
                sh = x
            if dj == 1:
                sh = jnp.where(col != HW - 1, sh, 0.0)
            elif dj == -1:
                sh = jnp.where(col != 0, sh, 0.0)
            acc = acc + w_ref[(di + 1) * 3 + (dj + 1)][None, :] * sh
    o_ref[0] = _silu(acc)


def _conv_silu(xc, conv_w9, conv_b):
    return pl.pallas_call(
        _conv_kernel,
        grid=(B,),
        in_specs=[
            pl.BlockSpec((1, L, DI), lambda b: (b, 0, 0)),
            pl.BlockSpec((9, DI), lambda b: (0, 0)),
            pl.BlockSpec((1, DI), lambda b: (0, 0)),
        ],
        out_specs=pl.BlockSpec((1, L, DI), lambda b: (b, 0, 0)),
        out_shape=jax.ShapeDtypeStruct((B, L, DI), jnp.float32),
        name="dwconv_silu",
        interpret=_INTERPRET,
    )(xc.reshape(B, L, DI), conv_w9, conv_b.reshape(1, DI))


T = 128
LOG2E = 1.4426950408889634


def _scan_loop(o_ref, dt_s, du_s, bc_s, a2_s, *, backward):
    order = tuple(reversed(range(T))) if backward else tuple(range(T))

    def body(c, h):
        cc = (L // T - 1 - c) if backward else c
        base = pl.multiple_of(cc * T, T)
        A2 = a2_s[...]
        dtc = dt_s[pl.ds(base, T), :]
        duc = du_s[pl.ds(base, T), :]
        bcT = bc_s[pl.ds(base, T), :].T
        rows = [None] * T
        for s in order:
            a = jnp.exp2(A2 * dtc[s:s + 1])
            x = bcT[:N, s:s + 1] * duc[s:s + 1]
            h = a * h + x
            rows[s] = jnp.sum(h * bcT[N:, s:s + 1], axis=0, keepdims=True)
        o_ref[0, 0, pl.ds(base, T), :] = jnp.concatenate(rows, axis=0)
        return h

    jax.lax.fori_loop(0, L // T, body, jnp.zeros((N, DI), jnp.float32))


def _scan_kernel(seq_ref, xw_ref, dtw_ref, dtb_ref, alog_ref, o_ref,
                 dt_s, du_s, bc_s, a2_s):
    seq = seq_ref[0, 0]
    xdbl = _dotT(seq, xw_ref[0])
    dtr = xdbl[:, :R]
    bc_s[...] = xdbl[:, R:]
    dt = _softplus(_dotT(dtr, dtw_ref[0]) + dtb_ref[0])
    dt_s[...] = dt
    du_s[...] = dt * seq
    a2_s[...] = -LOG2E * jnp.exp(alog_ref[0])

    j = pl.program_id(1)

    @pl.when(j % 2 == 0)
    def _():
        _scan_loop(o_ref, dt_s, du_s, bc_s, a2_s, backward=False)

    @pl.when(j % 2 == 1)
    def _():
        _scan_loop(o_ref, dt_s, du_s, bc_s, a2_s, backward=True)


def _scan(seqs, x_proj_w, dt_proj_w, dt_proj_b, a_log_t):
    wmap = lambda b, j: ((j % 2) * 2 + j // 2, 0, 0)
    return pl.pallas_call(
        _scan_kernel,
        grid=(B, 4),
        in_specs=[
            pl.BlockSpec((1, 1, L, DI), lambda b, j: (b, j // 2, 0, 0)),
            pl.BlockSpec((1, R + 2 * N, DI), wmap),
            pl.BlockSpec((1, DI, R), wmap),
            pl.BlockSpec((1, 1, DI), wmap),
            pl.BlockSpec((1, N, DI), wmap),
        ],
        out_specs=pl.BlockSpec((1, 1, L, DI), lambda b, j: (b, j, 0, 0)),
        out_shape=jax.ShapeDtypeStruct((B, 4, L, DI), jnp.float32),
        scratch_shapes=[
            pltpu.VMEM((L, DI), jnp.float32),
            pltpu.VMEM((L, DI), jnp.float32),
            pltpu.VMEM((L, 2 * N), jnp.float32),
            pltpu.VMEM((N, DI), jnp.float32),
        ],
        compiler_params=pltpu.CompilerParams(
            dimension_semantics=("arbitrary", "arbitrary"),
            vmem_limit_bytes=50 * 1024 * 1024,
        ),
        name="ss2d_scan",
        interpret=_INTERPRET,
    )(seqs, x_proj_w, dt_proj_w, dt_proj_b, a_log_t)


def _combine_kernel(yr_ref, yt_ref, xc_ref, ds_ref, z_ref, xh_ref, onw_ref,
                    onb_ref, ow_ref, o_ref):
    ds_sum = jnp.sum(ds_ref[...], axis=0, keepdims=True)
    y = yr_ref[...] + yt_ref[...] + ds_sum * xc_ref[...]
    y = _ln(y, onw_ref[...], onb_ref[...])
    y = y * _silu(z_ref[...])
    o_ref[...] = xh_ref[...] + _dotT(y, ow_ref[...])


def _combine(y_r, y_t, xc, ds, z, xh, onw, onb, ow):
    return pl.pallas_call(
        _combine_kernel,
        grid=(B,),
        in_specs=[
            pl.BlockSpec((L, DI), lambda b: (b, 0)),
            pl.BlockSpec((L, DI), lambda b: (b, 0)),
            pl.BlockSpec((L, DI), lambda b: (b, 0)),
            pl.BlockSpec((4, DI), lambda b: (0, 0)),
            pl.BlockSpec((L, DI), lambda b: (b, 0)),
            pl.BlockSpec((L, C), lambda b: (b, 0)),
            pl.BlockSpec((1, DI), lambda b: (0, 0)),
            pl.BlockSpec((1, DI), lambda b: (0, 0)),
            pl.BlockSpec((C, DI), lambda b: (0, 0)),
        ],
        out_specs=pl.BlockSpec((L, C), lambda b: (b, 0)),
        out_shape=jax.ShapeDtypeStruct((B * L, C), jnp.float32),
        name="combine_out",
        interpret=_INTERPRET,
    )(y_r, y_t, xc, ds, z, xh, onw.reshape(1, DI), onb.reshape(1, DI), ow)


def _spatial_t(a):
    return a.reshape(B, HW, HW, DI).transpose(0, 2, 1, 3).reshape(B, L, DI)


def kernel(x, skip, pe_expand_w, pe_norm_w, pe_norm_b, lp_w, lp_b,
           blk_ln_w, blk_ln_b, in_proj_w, conv_w, conv_b, x_proj_w,
           dt_proj_w, dt_proj_b, A_log, Ds, out_norm_w, out_norm_b,
           out_proj_w):
    H = x.shape[2]
    xf = x.transpose(0, 2, 3, 1).reshape(B * H * H, 2 * C)
    pe = _patch_expand(xf, pe_expand_w, pe_norm_w, pe_norm_b)
    xh_ps = (pe.reshape(B, H, H, 2, 2, C).transpose(0, 1, 3, 2, 4, 5)
             .reshape(B * L, C))
    skip_f = skip.transpose(0, 2, 3, 1).reshape(B * L, C)
    xh = _concat_proj(xh_ps, skip_f, lp_w, lp_b)

    depth = blk_ln_w.shape[0]
    for i in range(depth):
        xc, z = _ln_inproj(xh, blk_ln_w[i], blk_ln_b[i], in_proj_w[i])
        xc = _conv_silu(xc, conv_w[i].reshape(DI, 9).T, conv_b[i])
        seqs = jnp.stack([xc, _spatial_t(xc)], axis=1)
        ys = _scan(seqs, x_proj_w[i], dt_proj_w[i],
                   dt_proj_b[i].reshape(4, 1, DI),
                   A_log[i].transpose(0, 2, 1))
        y_r = (ys[:, 0] + ys[:, 1]).reshape(B * L, DI)
        y_t = _spatial_t(ys[:, 2] + ys[:, 3]).reshape(B * L, DI)
        xh = _combine(y_r, y_t, xc.reshape(B * L, DI), Ds[i], z, xh,
                      out_norm_w[i], out_norm_b[i], out_proj_w[i])

    return xh.reshape(B, HW, HW, C).transpose(0, 3, 1, 2)
